# baseline (speedup 1.0000x reference)
import dataclasses
import functools

import jax
import jax.numpy as jnp
from jax import lax
from jax.experimental import pallas as pl
from jax.experimental.pallas import tpu as pltpu
from jax.experimental.pallas import tpu_sc as plsc

_B, _F, _V, _D = 16384, 26, 100000, 16
_N = _B * _F
_H1, _H2 = 32, 32
_W = 1664
_WB = 64
_SUB = 128
_C = 2048
_R = 1024
_EP = 512


def _sc_params():
    cp = pltpu.CompilerParams()
    if "needs_layout_passes" in pltpu.CompilerParams.__dataclass_fields__:
        cp = dataclasses.replace(cp, needs_layout_passes=False)
    return cp


def _sc_fm_gather(fm128, g_idx, base16):
    mesh = plsc.VectorSubcoreMesh(core_axis_name="core",
                                  subcore_axis_name="subcore")

    @functools.partial(
        pl.kernel,
        compiler_params=_sc_params(),
        out_type=jax.ShapeDtypeStruct((_B, _EP), jnp.float32),
        mesh=mesh,
        scratch_types=[pltpu.VMEM((_SUB, 128), jnp.float32)],
    )
    def k(fm_hbm, g_hbm, b_hbm, emb_hbm, rows_v):
        ramp = lax.iota(jnp.int32, 16)

        def body(g_v, b_v, emb_v):
            for sc in range(_W // _SUB):
                pltpu.sync_copy(
                    fm_hbm.at[g_v.at[0, pl.ds(sc * _SUB, _SUB)]], rows_v)

                @pl.loop(sc * (_SUB // 16), (sc + 1) * (_SUB // 16))
                def _(t):
                    j = ramp + 16 * t
                    brow = j // _F
                    blane = (j - brow * _F) * _D
                    bt = b_v[0, pl.ds(16 * t, 16)]
                    for d in range(16):
                        vals = plsc.load_gather(
                            rows_v, [j - sc * _SUB, bt + d])
                        plsc.store_scatter(emb_v, [brow, blane + d], vals)

        pltpu.emit_pipeline(
            body,
            grid=(_N // _W,),
            in_specs=[
                pl.BlockSpec((1, _W), lambda i: (0, i)),
                pl.BlockSpec((1, _W), lambda i: (0, i)),
            ],
            out_specs=[
                pl.BlockSpec((_WB, _EP), lambda i: (i, 0)),
            ],
            core_axis_name=("core", "subcore"),
            dimension_semantics=(pltpu.PARALLEL,),
        )(g_hbm, b_hbm, emb_hbm)

    return k(fm128, g_idx, base16)


def _sc_linear(lin_flat, idxT):
    mesh = plsc.VectorSubcoreMesh(core_axis_name="core",
                                  subcore_axis_name="subcore")

    @functools.partial(
        pl.kernel,
        compiler_params=_sc_params(),
        out_type=jax.ShapeDtypeStruct((_F * _B,), jnp.float32),
        mesh=mesh,
        scratch_types=[
            pltpu.VMEM((_V,), jnp.float32),
            pltpu.VMEM((_C,), jnp.int32),
            pltpu.VMEM((_C,), jnp.float32),
        ],
    )
    def k(tab_hbm, idx_hbm, out_hbm, tab_v, lidx_v, val_v):
        w = lax.axis_index("core") * 16 + lax.axis_index("subcore")

        @pl.when(w < _F)
        def _():
            pltpu.sync_copy(tab_hbm.at[pl.ds(w * _V, _V)], tab_v)

            @pl.loop(0, _B // _C)
            def _(c):
                pltpu.sync_copy(idx_hbm.at[pl.ds(w * _B + c * _C, _C)], lidx_v)

                @pl.loop(0, _C // 16)
                def _(t):
                    iv = lidx_v[pl.ds(16 * t, 16)]
                    val_v[pl.ds(16 * t, 16)] = plsc.load_gather(tab_v, [iv])

                pltpu.sync_copy(val_v, out_hbm.at[pl.ds(w * _B + c * _C, _C)])

    return k(lin_flat, idxT)


def _tc_body(emb_ref, lin_ref, xv_ref, w1_ref, b1_ref, w2_ref, b2_ref,
             bias_ref, out_ref):
    xv = xv_ref[...]
    e = emb_ref[...][:, :_F * _D] * xv
    h = jnp.dot(e, w1_ref[...], preferred_element_type=jnp.float32)
    h = jnp.maximum(h + b1_ref[...], 0.0)
    h = jnp.dot(h, w2_ref[...], preferred_element_type=jnp.float32)
    h = jnp.maximum(h + b2_ref[...], 0.0)
    deep = jnp.sum(h, axis=1, keepdims=True)

    col = lax.broadcasted_iota(jnp.int32, (_F * _D, _D), 0)
    dd = lax.broadcasted_iota(jnp.int32, (_F * _D, _D), 1)
    a = jnp.where((col % _D) == dd, 1.0, 0.0)
    s = jnp.dot(e, a, preferred_element_type=jnp.float32)
    cross = 0.5 * (jnp.sum(s * s, axis=1, keepdims=True)
                   - jnp.sum(e * e, axis=1, keepdims=True))

    ones = jnp.full((_F, 1), 1.0, dtype=jnp.float32)
    lsum = lax.dot_general(lin_ref[...], ones, (((0,), (0,)), ((), ())),
                           preferred_element_type=jnp.float32)
    out_ref[...] = deep + cross + lsum * xv + bias_ref[...]


def _tc_dense(embP, linT, xv1, w1, b1, w2, b2, bias):
    return pl.pallas_call(
        _tc_body,
        grid=(_B // _R,),
        in_specs=[
            pl.BlockSpec((_R, _EP), lambda i: (i, 0)),
            pl.BlockSpec((_F, _R), lambda i: (0, i)),
            pl.BlockSpec((_R, 1), lambda i: (i, 0)),
            pl.BlockSpec((_F * _D, _H1), lambda i: (0, 0)),
            pl.BlockSpec((1, _H1), lambda i: (0, 0)),
            pl.BlockSpec((_H1, _H2), lambda i: (0, 0)),
            pl.BlockSpec((1, _H2), lambda i: (0, 0)),
            pl.BlockSpec((1, 1), lambda i: (0, 0)),
        ],
        out_specs=pl.BlockSpec((_R, 1), lambda i: (i, 0)),
        out_shape=jax.ShapeDtypeStruct((_B, 1), jnp.float32),
    )(embP, linT, xv1, w1, b1, w2, b2, bias)


def kernel(Xi, Xv, linear_tables, fm_tables, W1, b1, W2, b2, bias):
    idx = Xi[:, :, 0].astype(jnp.int32)
    offs = (jnp.arange(_F, dtype=jnp.int32) * _V)[None, :]
    fm_idx = idx + offs
    g_idx = (fm_idx // 8).reshape(1, _N)
    base16 = ((fm_idx % 8) * 16).reshape(1, _N)
    idxT = idx.T.reshape(_F * _B)

    fm128 = fm_tables.reshape((_F * _V) // 8, 128)
    lin_flat = linear_tables.reshape(_F * _V)

    embP = _sc_fm_gather(fm128, g_idx, base16)
    linT = _sc_linear(lin_flat, idxT)

    xv1 = Xv[:, 1:2]
    out = _tc_dense(embP, linT.reshape(_F, _B), xv1, W1,
                    b1.reshape(1, _H1), W2, b2.reshape(1, _H2),
                    bias.reshape(1, 1))
    return out[:, 0]


# speedup vs baseline: 1.0138x; 1.0138x over previous
import dataclasses
import functools

import jax
import jax.numpy as jnp
from jax import lax
from jax.experimental import pallas as pl
from jax.experimental.pallas import tpu as pltpu
from jax.experimental.pallas import tpu_sc as plsc

_B, _F, _V, _D = 16384, 26, 100000, 16
_N = _B * _F
_H1, _H2 = 32, 32
_W = 512
_C = 2048
_R = 1024


def _sc_params():
    cp = pltpu.CompilerParams()
    if "needs_layout_passes" in pltpu.CompilerParams.__dataclass_fields__:
        cp = dataclasses.replace(cp, needs_layout_passes=False)
    return cp


def _mesh():
    return plsc.VectorSubcoreMesh(core_axis_name="core",
                                  subcore_axis_name="subcore")


def _sc_fm_gather(fm128, g_idx, base16):
    @functools.partial(
        pl.kernel,
        compiler_params=_sc_params(),
        out_type=jax.ShapeDtypeStruct((_N * _D // 128, 128), jnp.float32),
        mesh=_mesh(),
        scratch_types=[pltpu.VMEM((_W, 128), jnp.float32)],
    )
    def k(fm_hbm, g_hbm, b_hbm, emb_hbm, rows_v):
        ramp = lax.iota(jnp.int32, 16)

        def body(g_v, b_v, emb_v):
            pltpu.sync_copy(fm_hbm.at[g_v.at[0]], rows_v)

            @pl.loop(0, _W // 16)
            def _(t):
                rows16 = ramp + 16 * t
                bt = b_v[0, pl.ds(16 * t, 16)]
                for d in range(16):
                    vals = plsc.load_gather(rows_v, [rows16, bt + d])
                    p = 256 * t + 16 * ramp + d
                    plsc.store_scatter(
                        emb_v,
                        [lax.shift_right_logical(p, 7), lax.bitwise_and(p, 127)],
                        vals)

        pltpu.emit_pipeline(
            body,
            grid=(_N // _W,),
            in_specs=[
                pl.BlockSpec((1, _W), lambda i: (0, i)),
                pl.BlockSpec((1, _W), lambda i: (0, i)),
            ],
            out_specs=[
                pl.BlockSpec((_W * _D // 128, 128), lambda i: (i, 0)),
            ],
            core_axis_name=("core", "subcore"),
            dimension_semantics=(pltpu.PARALLEL,),
        )(g_hbm, b_hbm, emb_hbm)

    return k(fm128, g_idx, base16)


_LSPAN = 800
_LROWS = 20336


def _sc_linear(lin128, idxT):
    @functools.partial(
        pl.kernel,
        compiler_params=_sc_params(),
        out_type=jax.ShapeDtypeStruct((_F * _B,), jnp.float32),
        mesh=_mesh(),
        scratch_types=[
            pltpu.VMEM((_LSPAN, 128), jnp.float32),
            pltpu.VMEM((_C,), jnp.int32),
            pltpu.VMEM((_C,), jnp.float32),
        ],
    )
    def k(tab_hbm, idx_hbm, out_hbm, tab_v, idx_v, val_v):
        w = lax.axis_index("core") * 16 + lax.axis_index("subcore")

        @pl.when(w < _F)
        def _():
            base_row = pl.multiple_of((w * _V // 128) // 8 * 8, 8)
            off = w * _V - base_row * 128
            pltpu.sync_copy(tab_hbm.at[pl.ds(base_row, _LSPAN)], tab_v)

            @pl.loop(0, _B // _C)
            def _(c):
                pltpu.sync_copy(idx_hbm.at[pl.ds(w * _B + c * _C, _C)], idx_v)

                @pl.loop(0, _C // 16)
                def _(t):
                    l = idx_v[pl.ds(16 * t, 16)] + off
                    val_v[pl.ds(16 * t, 16)] = plsc.load_gather(
                        tab_v,
                        [lax.shift_right_logical(l, 7),
                         lax.bitwise_and(l, 127)])

                pltpu.sync_copy(val_v, out_hbm.at[pl.ds(w * _B + c * _C, _C)])

    return k(lin128, idxT)


def _tc_body(emb_ref, lin_ref, xv_ref, w1_ref, b1_ref, w2_ref, b2_ref,
             bias_ref, out_ref):
    xv = xv_ref[...]
    e = emb_ref[...] * xv
    h = jnp.dot(e, w1_ref[...], preferred_element_type=jnp.float32)
    h = jnp.maximum(h + b1_ref[...], 0.0)
    h = jnp.dot(h, w2_ref[...], preferred_element_type=jnp.float32)
    h = jnp.maximum(h + b2_ref[...], 0.0)
    deep = jnp.sum(h, axis=1, keepdims=True)

    col = lax.broadcasted_iota(jnp.int32, (_F * _D, _D), 0)
    dd = lax.broadcasted_iota(jnp.int32, (_F * _D, _D), 1)
    a = jnp.where((col % _D) == dd, 1.0, 0.0)
    s = jnp.dot(e, a, preferred_element_type=jnp.float32)
    cross = 0.5 * (jnp.sum(s * s, axis=1, keepdims=True)
                   - jnp.sum(e * e, axis=1, keepdims=True))

    ones = jnp.full((_F, 1), 1.0, dtype=jnp.float32)
    lsum = lax.dot_general(lin_ref[...], ones, (((0,), (0,)), ((), ())),
                           preferred_element_type=jnp.float32)
    out_ref[...] = deep + cross + lsum * xv + bias_ref[...]


def _tc_dense(emb2, linT, xv1, w1, b1, w2, b2, bias):
    return pl.pallas_call(
        _tc_body,
        grid=(_B // _R,),
        in_specs=[
            pl.BlockSpec((_R, _F * _D), lambda i: (i, 0)),
            pl.BlockSpec((_F, _R), lambda i: (0, i)),
            pl.BlockSpec((_R, 1), lambda i: (i, 0)),
            pl.BlockSpec((_F * _D, _H1), lambda i: (0, 0)),
            pl.BlockSpec((1, _H1), lambda i: (0, 0)),
            pl.BlockSpec((_H1, _H2), lambda i: (0, 0)),
            pl.BlockSpec((1, _H2), lambda i: (0, 0)),
            pl.BlockSpec((1, 1), lambda i: (0, 0)),
        ],
        out_specs=pl.BlockSpec((_R, 1), lambda i: (i, 0)),
        out_shape=jax.ShapeDtypeStruct((_B, 1), jnp.float32),
    )(emb2, linT, xv1, w1, b1, w2, b2, bias)


def kernel(Xi, Xv, linear_tables, fm_tables, W1, b1, W2, b2, bias):
    idx = Xi[:, :, 0].astype(jnp.int32)
    offs = (jnp.arange(_F, dtype=jnp.int32) * _V)[None, :]
    fm_idx = idx + offs
    g_idx = (fm_idx // 8).reshape(1, _N)
    base16 = ((fm_idx % 8) * 16).reshape(1, _N)
    idxT = idx.T.reshape(_F * _B)

    fm128 = fm_tables.reshape((_F * _V) // 8, 128)
    lin128 = jnp.pad(linear_tables.reshape(_F * _V),
                     (0, _LROWS * 128 - _F * _V)).reshape(_LROWS, 128)

    embf = _sc_fm_gather(fm128, g_idx, base16)
    linT = _sc_linear(lin128, idxT).reshape(_F, _B)

    emb2 = embf.reshape(_B, _F * _D)
    xv1 = Xv[:, 1:2]
    out = _tc_dense(emb2, linT, xv1, W1, b1.reshape(1, _H1), W2,
                    b2.reshape(1, _H2), bias.reshape(1, 1))
    return out[:, 0]


# speedup vs baseline: 1.0170x; 1.0031x over previous
import dataclasses
import functools

import jax
import jax.numpy as jnp
from jax import lax
from jax.experimental import pallas as pl
from jax.experimental.pallas import tpu as pltpu
from jax.experimental.pallas import tpu_sc as plsc

_B, _F, _V, _D = 16384, 26, 100000, 16
_N = _B * _F
_H1, _H2 = 32, 32
_W = 512
_C = 2048
_R = 1024


def _sc_params():
    cp = pltpu.CompilerParams()
    if "needs_layout_passes" in pltpu.CompilerParams.__dataclass_fields__:
        cp = dataclasses.replace(cp, needs_layout_passes=False)
    return cp


def _mesh():
    return plsc.VectorSubcoreMesh(core_axis_name="core",
                                  subcore_axis_name="subcore")


def _sc_fm_gather(fm128, g_idx, base16):
    @functools.partial(
        pl.kernel,
        compiler_params=_sc_params(),
        out_type=jax.ShapeDtypeStruct((_N * _D // 128, 128), jnp.float32),
        mesh=_mesh(),
        scratch_types=[pltpu.VMEM((_W, 128), jnp.float32)],
    )
    def k(fm_hbm, g_hbm, b_hbm, emb_hbm, rows_v):
        ramp = lax.iota(jnp.int32, 16)

        def body(g_v, b_v, emb_v):
            pltpu.sync_copy(fm_hbm.at[g_v.at[0]], rows_v)

            @pl.loop(0, _W // 16)
            def _(t):
                rows16 = ramp + 16 * t
                bt = b_v[0, pl.ds(16 * t, 16)]
                for d in range(16):
                    vals = plsc.load_gather(rows_v, [rows16, bt + d])
                    p = 256 * t + 16 * ramp + d
                    plsc.store_scatter(
                        emb_v,
                        [lax.shift_right_logical(p, 7), lax.bitwise_and(p, 127)],
                        vals)

        pltpu.emit_pipeline(
            body,
            grid=(_N // _W,),
            in_specs=[
                pl.BlockSpec((1, _W), lambda i: (0, i)),
                pl.BlockSpec((1, _W), lambda i: (0, i)),
            ],
            out_specs=[
                pl.BlockSpec((_W * _D // 128, 128), lambda i: (i, 0)),
            ],
            core_axis_name=("core", "subcore"),
            dimension_semantics=(pltpu.PARALLEL,),
        )(g_hbm, b_hbm, emb_hbm)

    return k(fm128, g_idx, base16)


def _sc_linear(lin_flat, idxT):
    @functools.partial(
        pl.kernel,
        compiler_params=_sc_params(),
        out_type=jax.ShapeDtypeStruct((_F * _B,), jnp.float32),
        mesh=_mesh(),
        scratch_types=[
            pltpu.VMEM((_V,), jnp.float32),
            pltpu.VMEM((_C,), jnp.int32),
            pltpu.VMEM((_C,), jnp.float32),
        ],
    )
    def k(tab_hbm, idx_hbm, out_hbm, tab_v, idx_v, val_v):
        ramp = lax.iota(jnp.int32, 16)
        w = lax.axis_index("core") * 16 + lax.axis_index("subcore")

        @pl.when(w < _F)
        def _():
            pltpu.sync_copy(tab_hbm.at[pl.ds(w * _V, _V)], tab_v)

            @pl.loop(0, _B // _C)
            def _(c):
                pltpu.sync_copy(idx_hbm.at[pl.ds(w * _B + c * _C, _C)], idx_v)

                @pl.loop(0, _C // 16)
                def _(t):
                    iv = idx_v[pl.ds(16 * t, 16)]
                    val_v[pl.ds(16 * t, 16)] = plsc.load_gather(tab_v, [iv])

                pltpu.sync_copy(val_v, out_hbm.at[pl.ds(w * _B + c * _C, _C)])

    return k(lin_flat, idxT)


def _tc_body(emb_ref, lin_ref, xv_ref, w1_ref, b1_ref, w2_ref, b2_ref,
             bias_ref, out_ref):
    xv = xv_ref[...]
    e = emb_ref[...] * xv
    h = jnp.dot(e, w1_ref[...], preferred_element_type=jnp.float32)
    h = jnp.maximum(h + b1_ref[...], 0.0)
    h = jnp.dot(h, w2_ref[...], preferred_element_type=jnp.float32)
    h = jnp.maximum(h + b2_ref[...], 0.0)
    deep = jnp.sum(h, axis=1, keepdims=True)

    col = lax.broadcasted_iota(jnp.int32, (_F * _D, _D), 0)
    dd = lax.broadcasted_iota(jnp.int32, (_F * _D, _D), 1)
    a = jnp.where((col % _D) == dd, 1.0, 0.0)
    s = jnp.dot(e, a, preferred_element_type=jnp.float32)
    cross = 0.5 * (jnp.sum(s * s, axis=1, keepdims=True)
                   - jnp.sum(e * e, axis=1, keepdims=True))

    ones = jnp.full((_F, 1), 1.0, dtype=jnp.float32)
    lsum = lax.dot_general(lin_ref[...], ones, (((0,), (0,)), ((), ())),
                           preferred_element_type=jnp.float32)
    out_ref[...] = deep + cross + lsum * xv + bias_ref[...]


def _tc_dense(emb2, linT, xv1, w1, b1, w2, b2, bias):
    return pl.pallas_call(
        _tc_body,
        grid=(_B // _R,),
        in_specs=[
            pl.BlockSpec((_R, _F * _D), lambda i: (i, 0)),
            pl.BlockSpec((_F, _R), lambda i: (0, i)),
            pl.BlockSpec((_R, 1), lambda i: (i, 0)),
            pl.BlockSpec((_F * _D, _H1), lambda i: (0, 0)),
            pl.BlockSpec((1, _H1), lambda i: (0, 0)),
            pl.BlockSpec((_H1, _H2), lambda i: (0, 0)),
            pl.BlockSpec((1, _H2), lambda i: (0, 0)),
            pl.BlockSpec((1, 1), lambda i: (0, 0)),
        ],
        out_specs=pl.BlockSpec((_R, 1), lambda i: (i, 0)),
        out_shape=jax.ShapeDtypeStruct((_B, 1), jnp.float32),
    )(emb2, linT, xv1, w1, b1, w2, b2, bias)


def kernel(Xi, Xv, linear_tables, fm_tables, W1, b1, W2, b2, bias):
    idx = Xi[:, :, 0].astype(jnp.int32)
    offs = (jnp.arange(_F, dtype=jnp.int32) * _V)[None, :]
    fm_idx = idx + offs
    g_idx = (fm_idx // 8).reshape(1, _N)
    base16 = ((fm_idx % 8) * 16).reshape(1, _N)
    idxT = idx.T.reshape(_F * _B)

    fm128 = fm_tables.reshape((_F * _V) // 8, 128)
    lin_flat = linear_tables.reshape(_F * _V)

    embf = _sc_fm_gather(fm128, g_idx, base16)
    linT = _sc_linear(lin_flat, idxT).reshape(_F, _B)

    emb2 = embf.reshape(_B, _F * _D)
    xv1 = Xv[:, 1:2]
    out = _tc_dense(emb2, linT, xv1, W1, b1.reshape(1, _H1), W2,
                    b2.reshape(1, _H2), bias.reshape(1, 1))
    return out[:, 0]
